# baseline (speedup 1.0000x reference)
import functools

import jax
import jax.numpy as jnp
from jax import lax
from jax.experimental import pallas as pl
from jax.experimental.pallas import tpu as pltpu
from jax.experimental.pallas import tpu_sc as plsc

TM = 256
K = 8192
C = 32

_NC, _NS = 2, 16
_NW = _NC * _NS
_M_TOTAL = 4608
_BPW = _M_TOTAL // _NW
_CHUNK = 72


_HW = 576
_NCHUNK = _HW // _BPW


def _sc_epilogue(table_hbm, idx_hbm, fthr_hbm, zq_hbm, di_hbm,
                 idx_v, rows_v, fthr_v, zqt_v, dit_v, sem):
    wid = lax.axis_index("s") * _NC + lax.axis_index("c")
    n = wid // _NCHUNK
    hwb = (wid % _NCHUNK) * _BPW
    tokbase = n * _HW + hwb
    pltpu.sync_copy(fthr_hbm.at[n, :, pl.ds(hwb, _BPW)], fthr_v)
    for j in range(_BPW // _CHUNK):
        pltpu.sync_copy(idx_hbm.at[pl.ds(tokbase + j * _CHUNK, _CHUNK)],
                        idx_v.at[j])
        pltpu.async_copy(table_hbm.at[idx_v.at[j]],
                         rows_v.at[pl.ds(j * _CHUNK, _CHUNK)], sem).wait()

    lane = lax.iota(jnp.int32, 16)

    def cbody(c, carry):
        cvec = jnp.zeros((16,), jnp.int32) + c
        for g in range(_BPW // 16):
            val = plsc.load_gather(rows_v, [g * 16 + lane, cvec])
            f = fthr_v[c, pl.ds(g * 16, 16)]
            zqt_v[c, pl.ds(g * 16, 16)] = val
            dit_v[c, pl.ds(g * 16, 16)] = f + (val - f)
        return carry

    lax.fori_loop(0, C, cbody, 0)
    pltpu.sync_copy(zqt_v, zq_hbm.at[n, :, pl.ds(hwb, _BPW)])
    pltpu.sync_copy(dit_v, di_hbm.at[n, :, pl.ds(hwb, _BPW)])


_sc_epilogue_call = functools.partial(
    pl.kernel,
    mesh=plsc.VectorSubcoreMesh(core_axis_name="c", subcore_axis_name="s"),
    out_type=(
        jax.ShapeDtypeStruct((8, C, _HW), jnp.float32),
        jax.ShapeDtypeStruct((8, C, _HW), jnp.float32),
    ),
    scratch_types=[
        pltpu.VMEM((_BPW // _CHUNK, _CHUNK), jnp.int32),
        pltpu.VMEM((_BPW, C), jnp.float32),
        pltpu.VMEM((C, _BPW), jnp.float32),
        pltpu.VMEM((C, _BPW), jnp.float32),
        pltpu.VMEM((C, _BPW), jnp.float32),
        pltpu.SemaphoreType.DMA,
    ],
    compiler_params=pltpu.CompilerParams(use_tc_tiling_on_sc=False,
                                         needs_layout_passes=False),
)(_sc_epilogue)


def _vq_body(zm2_ref, zsq_ref, esq_ref, emb_ref, idx_ref):
    zm2 = zm2_ref[...]
    zsq = zsq_ref[...]
    q = lax.dot_general(zm2, emb_ref[...], (((1,), (1,)), ((), ())),
                        preferred_element_type=jnp.float32)
    dist = (zsq + q) + esq_ref[...]
    mv = jnp.min(dist, axis=1, keepdims=True)
    gidx = lax.broadcasted_iota(jnp.int32, (TM, K), 1)
    idx = jnp.min(jnp.where(dist == mv, gidx, jnp.int32(K)), axis=1)
    idx_ref[...] = idx.reshape(1, 1, TM)


def kernel(feather, embedding):
    N, Cc, H, W = feather.shape
    z = jnp.transpose(feather, (0, 2, 3, 1)).reshape(-1, Cc)
    M = z.shape[0]
    zsq = jnp.sum(z * z, axis=1, keepdims=True)
    esq = jnp.sum(embedding * embedding, axis=1)[None, :]
    zm2 = -2.0 * z

    nearest_blocks = pl.pallas_call(
        _vq_body,
        grid=(M // TM,),
        in_specs=[
            pl.BlockSpec((TM, Cc), lambda i: (i, 0)),
            pl.BlockSpec((TM, 1), lambda i: (i, 0)),
            pl.BlockSpec((1, K), lambda i: (0, 0)),
            pl.BlockSpec((K, Cc), lambda i: (0, 0)),
        ],
        out_specs=pl.BlockSpec((1, 1, TM), lambda i: (i, 0, 0)),
        out_shape=jax.ShapeDtypeStruct((M // TM, 1, TM), jnp.int32),
    )(zm2, zsq, esq, embedding)

    nearest_flat = nearest_blocks.reshape(M)
    fthr_r = feather.reshape(N, Cc, H * W)
    zq_r, di_r = _sc_epilogue_call(embedding, nearest_flat, fthr_r)
    nearest = nearest_flat.reshape(N, 1, H, W)
    zq = zq_r.reshape(N, Cc, H, W)
    decoder_input = di_r.reshape(N, Cc, H, W)
    return decoder_input, zq, nearest


# speedup vs baseline: 1.1381x; 1.1381x over previous
import functools

import jax
import jax.numpy as jnp
from jax import lax
from jax.experimental import pallas as pl
from jax.experimental.pallas import tpu as pltpu
from jax.experimental.pallas import tpu_sc as plsc

TM = 512
K = 8192
C = 32

_NC, _NS = 2, 16
_NW = _NC * _NS
_M_TOTAL = 4608
_BPW = _M_TOTAL // _NW
_CHUNK = 72


def _sc_gather(table_hbm, idx_hbm, out_hbm, idx_v, rows_v, sem):
    wid = lax.axis_index("s") * _NC + lax.axis_index("c")
    base = wid * _BPW
    for j in range(_BPW // _CHUNK):
        pltpu.sync_copy(idx_hbm.at[pl.ds(base + j * _CHUNK, _CHUNK)],
                        idx_v.at[j])
        pltpu.async_copy(table_hbm.at[idx_v.at[j]],
                         rows_v.at[pl.ds(j * _CHUNK, _CHUNK)], sem).wait()
    pltpu.sync_copy(rows_v, out_hbm.at[pl.ds(base, _BPW)])


_sc_gather_call = functools.partial(
    pl.kernel,
    mesh=plsc.VectorSubcoreMesh(core_axis_name="c", subcore_axis_name="s"),
    out_type=jax.ShapeDtypeStruct((_M_TOTAL, C), jnp.float32),
    scratch_types=[
        pltpu.VMEM((_BPW // _CHUNK, _CHUNK), jnp.int32),
        pltpu.VMEM((_BPW, C), jnp.float32),
        pltpu.SemaphoreType.DMA,
    ],
    compiler_params=pltpu.CompilerParams(use_tc_tiling_on_sc=False),
)(_sc_gather)


def _vq_body(zm2_ref, zsq_ref, esq_ref, emb_ref, idx_ref):
    zm2 = zm2_ref[...]
    zsq = zsq_ref[...]
    q = lax.dot_general(zm2, emb_ref[...], (((1,), (1,)), ((), ())),
                        preferred_element_type=jnp.float32)
    dist = (zsq + q) + esq_ref[...]
    mv = jnp.min(dist, axis=1, keepdims=True)
    gidx = lax.broadcasted_iota(jnp.int32, (TM, K), 1)
    idx = jnp.min(jnp.where(dist == mv, gidx, jnp.int32(K)), axis=1)
    idx_ref[...] = idx.reshape(1, 1, TM)


def kernel(feather, embedding):
    N, Cc, H, W = feather.shape
    z = jnp.transpose(feather, (0, 2, 3, 1)).reshape(-1, Cc)
    M = z.shape[0]
    zsq = jnp.sum(z * z, axis=1, keepdims=True)
    esq = jnp.sum(embedding * embedding, axis=1)[None, :]
    zm2 = -2.0 * z

    nearest_blocks = pl.pallas_call(
        _vq_body,
        grid=(M // TM,),
        in_specs=[
            pl.BlockSpec((TM, Cc), lambda i: (i, 0)),
            pl.BlockSpec((TM, 1), lambda i: (i, 0)),
            pl.BlockSpec((1, K), lambda i: (0, 0)),
            pl.BlockSpec((K, Cc), lambda i: (0, 0)),
        ],
        out_specs=pl.BlockSpec((1, 1, TM), lambda i: (i, 0, 0)),
        out_shape=jax.ShapeDtypeStruct((M // TM, 1, TM), jnp.int32),
    )(zm2, zsq, esq, embedding)

    nearest_flat = nearest_blocks.reshape(M)
    zq_flat = _sc_gather_call(embedding, nearest_flat)
    nearest = nearest_flat.reshape(N, 1, H, W)
    zq = jnp.transpose(zq_flat.reshape(N, H, W, Cc), (0, 3, 1, 2))
    decoder_input = feather + lax.stop_gradient(zq - feather)
    return decoder_input, zq, nearest


# speedup vs baseline: 1.1656x; 1.0242x over previous
import functools

import jax
import jax.numpy as jnp
from jax import lax
from jax.experimental import pallas as pl
from jax.experimental.pallas import tpu as pltpu
from jax.experimental.pallas import tpu_sc as plsc

TM = 576
K = 8192
C = 32

_NC, _NS = 2, 16
_NW = _NC * _NS
_M_TOTAL = 4608
_BPW = _M_TOTAL // _NW
_CHUNK = 72


def _sc_gather(table_hbm, idx_hbm, out_hbm, idx_v, rows_v, sem):
    wid = lax.axis_index("s") * _NC + lax.axis_index("c")
    base = wid * _BPW
    for j in range(_BPW // _CHUNK):
        pltpu.sync_copy(idx_hbm.at[pl.ds(base + j * _CHUNK, _CHUNK)],
                        idx_v.at[j])
        pltpu.async_copy(table_hbm.at[idx_v.at[j]],
                         rows_v.at[pl.ds(j * _CHUNK, _CHUNK)], sem).wait()
    pltpu.sync_copy(rows_v, out_hbm.at[pl.ds(base, _BPW)])


_sc_gather_call = functools.partial(
    pl.kernel,
    mesh=plsc.VectorSubcoreMesh(core_axis_name="c", subcore_axis_name="s"),
    out_type=jax.ShapeDtypeStruct((_M_TOTAL, C), jnp.float32),
    scratch_types=[
        pltpu.VMEM((_BPW // _CHUNK, _CHUNK), jnp.int32),
        pltpu.VMEM((_BPW, C), jnp.float32),
        pltpu.SemaphoreType.DMA,
    ],
    compiler_params=pltpu.CompilerParams(use_tc_tiling_on_sc=False),
)(_sc_gather)


def _vq_body(fthr_ref, esq_ref, emb_ref, idx_ref):
    f = fthr_ref[0]
    z = jnp.transpose(f, (1, 0))
    zm2 = -2.0 * z
    zsq = jnp.sum(z * z, axis=1, keepdims=True)
    q = lax.dot_general(zm2, emb_ref[...], (((1,), (1,)), ((), ())),
                        preferred_element_type=jnp.float32)
    dist = (zsq + q) + esq_ref[...]
    mv = jnp.min(dist, axis=1, keepdims=True)
    gidx = lax.broadcasted_iota(jnp.int32, (TM, K), 1)
    idx = jnp.min(jnp.where(dist == mv, gidx, jnp.int32(K)), axis=1)
    idx_ref[...] = idx.reshape(1, 1, TM)


def kernel(feather, embedding):
    N, Cc, H, W = feather.shape
    M = N * H * W
    fthr_r = feather.reshape(N, Cc, H * W)
    esq = jnp.sum(embedding * embedding, axis=1)[None, :]

    nearest_blocks = pl.pallas_call(
        _vq_body,
        grid=(M // TM,),
        in_specs=[
            pl.BlockSpec((1, Cc, TM), lambda i: (i, 0, 0)),
            pl.BlockSpec((1, K), lambda i: (0, 0)),
            pl.BlockSpec((K, Cc), lambda i: (0, 0)),
        ],
        out_specs=pl.BlockSpec((1, 1, TM), lambda i: (i, 0, 0)),
        out_shape=jax.ShapeDtypeStruct((M // TM, 1, TM), jnp.int32),
    )(fthr_r, esq, embedding)

    nearest_flat = nearest_blocks.reshape(M)
    zq_flat = _sc_gather_call(embedding, nearest_flat)
    nearest = nearest_flat.reshape(N, 1, H, W)
    zq = jnp.transpose(zq_flat.reshape(N, H, W, Cc), (0, 3, 1, 2))
    decoder_input = feather + lax.stop_gradient(zq - feather)
    return decoder_input, zq, nearest
